# baseline (speedup 1.0000x reference)
import functools

import jax
import jax.numpy as jnp
from jax import lax
from jax.experimental import pallas as pl
from jax.experimental.pallas import tpu as pltpu
from jax.experimental.pallas import tpu_sc as plsc

NC = 2
NS = 16
NW = NC * NS
GRP = 128
GPC = 4
CHUNK = GRP * GPC


def _edge_mlp(e_r, s_p, BD1, b1t, BD2, b2t, W3s, b3t, LAT, blk):
    R, PK = e_r.shape
    E = R * 8
    DOUT = W3s.shape[1] // 8

    def body(e_ref, s_ref, w1, bb1, w2, bb2, w3, bb3, o_ref):
        x1 = jnp.maximum(
            jnp.dot(e_ref[...], w1[...], preferred_element_type=jnp.float32)
            + bb1[...] + s_ref[...],
            0.0,
        )
        x2 = jnp.maximum(
            jnp.dot(x1, w2[...], preferred_element_type=jnp.float32) + bb2[...],
            0.0,
        )
        z = jnp.dot(x2, w3[...], preferred_element_type=jnp.float32) + bb3[...]
        sg = jax.nn.sigmoid(z)
        for c in range(8):
            o_ref[:, c, :] = sg[:, c * DOUT:(c + 1) * DOUT]

    return pl.pallas_call(
        body,
        grid=(R // blk,),
        in_specs=[
            pl.BlockSpec((blk, PK), lambda i: (i, 0)),
            pl.BlockSpec((blk, PK), lambda i: (i, 0)),
            pl.BlockSpec((PK, PK), lambda i: (0, 0)),
            pl.BlockSpec((1, PK), lambda i: (0, 0)),
            pl.BlockSpec((PK, PK), lambda i: (0, 0)),
            pl.BlockSpec((1, PK), lambda i: (0, 0)),
            pl.BlockSpec((PK, 8 * DOUT), lambda i: (0, 0)),
            pl.BlockSpec((1, 8 * DOUT), lambda i: (0, 0)),
        ],
        out_specs=pl.BlockSpec((blk, 8, DOUT), lambda i: (i, 0, 0)),
        out_shape=jax.ShapeDtypeStruct((R, 8, DOUT), jnp.float32),
    )(e_r, s_p, BD1, b1t.reshape(1, PK), BD2, b2t.reshape(1, PK),
      W3s, b3t.reshape(1, 8 * DOUT)).reshape(E, DOUT)


def _node_pre(n_feats, W1e_s, W1e_d, blk):
    N, DIN = n_feats.shape
    LAT = W1e_s.shape[1]

    def body(n_ref, ws_ref, wd_ref, p_ref, q_ref):
        x = n_ref[...]
        p_ref[...] = jnp.dot(x, ws_ref[...], preferred_element_type=jnp.float32)
        q_ref[...] = jnp.dot(x, wd_ref[...], preferred_element_type=jnp.float32)

    return pl.pallas_call(
        body,
        grid=(N // blk,),
        in_specs=[
            pl.BlockSpec((blk, DIN), lambda i: (i, 0)),
            pl.BlockSpec((DIN, LAT), lambda i: (0, 0)),
            pl.BlockSpec((DIN, LAT), lambda i: (0, 0)),
        ],
        out_specs=[
            pl.BlockSpec((blk, LAT), lambda i: (i, 0)),
            pl.BlockSpec((blk, LAT), lambda i: (i, 0)),
        ],
        out_shape=[
            jax.ShapeDtypeStruct((N, LAT), jnp.float32),
            jax.ShapeDtypeStruct((N, LAT), jnp.float32),
        ],
    )(n_feats, W1e_s, W1e_d)


def _sc_gather_add(ei_r, efr, P, Q, E):
    LAT = P.shape[1]
    PK = LAT * 8
    R = E // 8
    RPC = CHUNK // 8
    n_chunks = E // CHUNK
    iters = -(-n_chunks // NW)
    mesh = plsc.VectorSubcoreMesh(core_axis_name="c", subcore_axis_name="s")

    @functools.partial(
        pl.kernel,
        out_type=[
            jax.ShapeDtypeStruct((R, PK), jnp.float32),
            jax.ShapeDtypeStruct((R, PK), jnp.float32),
        ],
        mesh=mesh,
        scratch_types=[
            pltpu.VMEM((GPC, GRP), jnp.int32),
            pltpu.VMEM((GPC, GRP), jnp.int32),
            pltpu.VMEM((GPC, GRP), jnp.int32),
            pltpu.VMEM((GPC, GRP), jnp.int32),
            pltpu.VMEM((RPC, PK), jnp.float32),
            pltpu.VMEM((RPC, PK), jnp.float32),
            pltpu.VMEM((CHUNK, LAT), jnp.float32),
            pltpu.VMEM((CHUNK, LAT), jnp.float32),
            pltpu.VMEM((CHUNK, LAT), jnp.float32),
            pltpu.VMEM((CHUNK, LAT), jnp.float32),
            pltpu.SemaphoreType.DMA,
            pltpu.SemaphoreType.DMA,
            pltpu.SemaphoreType.DMA,
            pltpu.SemaphoreType.DMA,
            pltpu.SemaphoreType.DMA,
        ],
        compiler_params=pltpu.CompilerParams(use_tc_tiling_on_sc=False),
    )
    def k(ei_hbm, efr_hbm, p_hbm, q_hbm, s_hbm, er_hbm, isrc0, isrc1, idst0, idst1,
          sb0, sb1, pb0, pb1, qb0, qb1, semI, semG, semW0, semW1, semE):
        wid = lax.axis_index("s") * NC + lax.axis_index("c")
        slab = R // NW
        esl = pl.ds(wid * slab, slab)
        pltpu.async_copy(efr_hbm.at[esl], er_hbm.at[esl], semE)
        isrc = [isrc0, isrc1]
        idst = [idst0, idst1]
        sb = [sb0, sb1]
        pb = [pb0, pb1]
        qb = [qb0, qb1]
        semW = [semW0, semW1]

        def issue_idx(cid, b):
            g0 = cid * GPC
            pltpu.async_copy(ei_hbm.at[0, pl.ds(g0, GPC)], isrc[b], semI)
            pltpu.async_copy(ei_hbm.at[1, pl.ds(g0, GPC)], idst[b], semI)

        @pl.when(wid < n_chunks)
        def _():
            issue_idx(wid, 0)

        def outer(io, carry):
            for b in range(2):
                i = io * 2 + b
                cid = wid + i * NW

                @pl.when(cid < n_chunks)
                def _(i=i, b=b, cid=cid):
                    g0 = cid * GPC
                    pltpu.make_async_copy(
                        ei_hbm.at[0, pl.ds(g0, GPC)], isrc[b], semI).wait()
                    pltpu.make_async_copy(
                        ei_hbm.at[1, pl.ds(g0, GPC)], idst[b], semI).wait()
                    ncid = cid + NW

                    @pl.when(ncid < n_chunks)
                    def _():
                        issue_idx(ncid, 1 - b)

                    gs = []
                    for g in range(GPC):
                        sl = pl.ds(g * GRP, GRP)
                        gs.append(pltpu.async_copy(
                            p_hbm.at[isrc[b].at[g]], pb[b].at[sl], semG))
                        gs.append(pltpu.async_copy(
                            q_hbm.at[idst[b].at[g]], qb[b].at[sl], semG))
                    for h in gs:
                        h.wait()

                    @pl.when(i >= 2)
                    def _():
                        pltpu.make_async_copy(
                            s_hbm.at[pl.ds(0, RPC)], sb[b], semW[b]).wait()

                    def rows(j, c2):
                        for u in range(8):
                            r = j * 8 + u
                            sb[b][j, pl.ds(u * LAT, LAT)] = pb[b][r] + qb[b][r]
                        return c2

                    lax.fori_loop(0, RPC, rows, 0)
                    pltpu.async_copy(
                        sb[b], s_hbm.at[pl.ds(cid * RPC, RPC)], semW[b])

            return carry

        lax.fori_loop(0, (iters + 1) // 2, outer, 0)
        for b in range(2):
            pltpu.make_async_copy(s_hbm.at[pl.ds(0, RPC)], sb[b], semW[b]).wait()
        pltpu.make_async_copy(efr_hbm.at[esl], er_hbm.at[esl], semE).wait()

    return k(ei_r, efr, P, Q)


def _sc_segment_sum(ei_r, e_out, N, LAT):
    E, DOUT = e_out.shape
    HALF = DOUT // NC
    GPC2 = 2
    CH2 = GPC2 * GRP
    n_chunks = E // CH2
    iters = -(-n_chunks // NS)
    rs = (N // NS) // 8 * 8
    rs_last = N - (NS - 1) * rs
    ZR = 16
    nz = N // ZR
    mesh = plsc.VectorSubcoreMesh(core_axis_name="c", subcore_axis_name="s")

    @functools.partial(
        pl.kernel,
        out_type=[
            jax.ShapeDtypeStruct((N, DOUT), jnp.float32),
            jax.ShapeDtypeStruct((NC, N, LAT), jnp.float32),
        ],
        mesh=mesh,
        scratch_types=[
            pltpu.VMEM((GPC2, GRP), jnp.int32),
            pltpu.VMEM((GPC2, GRP), jnp.int32),
            pltpu.VMEM((CH2, HALF), jnp.float32),
            pltpu.VMEM((CH2, HALF), jnp.float32),
            pltpu.VMEM((GRP, LAT), jnp.float32),
            pltpu.VMEM((ZR, HALF), jnp.float32),
            pltpu.VMEM((ZR, LAT), jnp.float32),
            pltpu.VMEM_SHARED((N, HALF), jnp.float32),
            pltpu.VMEM_SHARED((N, LAT), jnp.float32),
            pltpu.SemaphoreType.DMA,
        ],
        compiler_params=pltpu.CompilerParams(use_tc_tiling_on_sc=False),
    )
    def k(ei_hbm, eout_hbm, sums_hbm, deg_hbm,
          idxb0, idxb1, rows0, rows1, onesb, zbs, zbd, ssh, dsh, sem):
        c = lax.axis_index("c")
        s = lax.axis_index("s")
        idxb = [idxb0, idxb1]
        rows = [rows0, rows1]

        zero16 = jnp.zeros((16,), jnp.float32)
        one16 = jnp.ones((16,), jnp.float32)
        for r in range(ZR):
            for c8 in range(HALF // 16):
                zbs[r, pl.ds(c8 * 16, 16)] = zero16
            for c8 in range(LAT // 16):
                zbd[r, pl.ds(c8 * 16, 16)] = zero16
        for r in range(GRP):
            for c8 in range(LAT // 16):
                onesb[r, pl.ds(c8 * 16, 16)] = one16

        def zbody(i, carry):
            j = s + i * NS

            @pl.when(j < nz)
            def _():
                pltpu.sync_copy(zbs, ssh.at[pl.ds(j * ZR, ZR)])
                pltpu.sync_copy(zbd, dsh.at[pl.ds(j * ZR, ZR)])

            return carry

        lax.fori_loop(0, -(-nz // NS), zbody, 0)
        plsc.subcore_barrier()

        def issue_reads(cid, b):
            g0 = cid * GPC2
            pltpu.async_copy(ei_hbm.at[1, pl.ds(g0, GPC2)], idxb[b], sem)
            pltpu.async_copy(
                eout_hbm.at[pl.ds(cid * CH2, CH2), pl.ds(c * HALF, HALF)],
                rows[b], sem)

        @pl.when(s < n_chunks)
        def _():
            issue_reads(s, 0)

        def outer(io, carry):
            for b in range(2):
                i = io * 2 + b
                cid = s + i * NS

                @pl.when(cid < n_chunks)
                def _(i=i, b=b, cid=cid):
                    g0 = cid * GPC2
                    pltpu.make_async_copy(
                        ei_hbm.at[1, pl.ds(g0, GPC2)], idxb[b], sem).wait()
                    pltpu.make_async_copy(
                        eout_hbm.at[pl.ds(cid * CH2, CH2),
                                    pl.ds(c * HALF, HALF)],
                        rows[b], sem).wait()
                    ncid = cid + NS

                    @pl.when(ncid < n_chunks)
                    def _():
                        issue_reads(ncid, 1 - b)

                    for g in range(GPC2):
                        sl = pl.ds(g * GRP, GRP)
                        pltpu.sync_copy(rows[b].at[sl], ssh.at[idxb[b].at[g]],
                                        add=True)

                    @pl.when(lax.rem(cid, NC) == c)
                    def _():
                        for g in range(GPC2):
                            pltpu.sync_copy(onesb, dsh.at[idxb[b].at[g]],
                                            add=True)

            return carry

        lax.fori_loop(0, (iters + 1) // 2, outer, 0)
        plsc.subcore_barrier()

        @pl.when(s < NS - 1)
        def _():
            pltpu.sync_copy(ssh.at[pl.ds(s * rs, rs)],
                            sums_hbm.at[pl.ds(s * rs, rs), pl.ds(c * HALF, HALF)])

        @pl.when(s == NS - 1)
        def _():
            base = (NS - 1) * rs
            pltpu.sync_copy(ssh.at[pl.ds(base, rs_last)],
                            sums_hbm.at[pl.ds(base, rs_last), pl.ds(c * HALF, HALF)])

        @pl.when(s < NS - 1)
        def _():
            pltpu.sync_copy(dsh.at[pl.ds(s * rs, rs)],
                            deg_hbm.at[c, pl.ds(s * rs, rs)])

        @pl.when(s == NS - 1)
        def _():
            base = (NS - 1) * rs
            pltpu.sync_copy(dsh.at[pl.ds(base, rs_last)],
                            deg_hbm.at[c, pl.ds(base, rs_last)])

    return k(ei_r, e_out)


def _node_mlp(sums_p, deg_p, n_feats, W1n_a, W1n_b, b1n, W2n, b2n, W3n, b3n, blk):
    N, DIN = n_feats.shape
    LAT = W1n_a.shape[1]
    DOUT = W3n.shape[1]

    def body(sp, dp, nf, w1a, w1b, b1, w2, b2, w3, b3, o_ref):
        sums = sp[...]
        deg = dp[0] + dp[1]
        hN = sums / jnp.maximum(deg[:, :1], 1.0)
        h = jnp.maximum(
            jnp.dot(nf[...], w1a[...], preferred_element_type=jnp.float32)
            + jnp.dot(hN, w1b[...], preferred_element_type=jnp.float32)
            + b1[...],
            0.0,
        )
        h = jnp.maximum(
            jnp.dot(h, w2[...], preferred_element_type=jnp.float32) + b2[...], 0.0
        )
        o_ref[...] = jax.nn.sigmoid(
            jnp.dot(h, w3[...], preferred_element_type=jnp.float32) + b3[...]
        )

    DSUM = sums_p.shape[1]
    DDEG = deg_p.shape[2]
    return pl.pallas_call(
        body,
        grid=(N // blk,),
        in_specs=[
            pl.BlockSpec((blk, DSUM), lambda i: (i, 0)),
            pl.BlockSpec((NC, blk, DDEG), lambda i: (0, i, 0)),
            pl.BlockSpec((blk, DIN), lambda i: (i, 0)),
            pl.BlockSpec((DIN, LAT), lambda i: (0, 0)),
            pl.BlockSpec((DSUM, LAT), lambda i: (0, 0)),
            pl.BlockSpec((1, LAT), lambda i: (0, 0)),
            pl.BlockSpec((LAT, LAT), lambda i: (0, 0)),
            pl.BlockSpec((1, LAT), lambda i: (0, 0)),
            pl.BlockSpec((LAT, DOUT), lambda i: (0, 0)),
            pl.BlockSpec((1, DOUT), lambda i: (0, 0)),
        ],
        out_specs=pl.BlockSpec((blk, DOUT), lambda i: (i, 0)),
        out_shape=jax.ShapeDtypeStruct((N, DOUT), jnp.float32),
    )(sums_p, deg_p, n_feats, W1n_a, W1n_b, b1n.reshape(1, LAT), W2n,
      b2n.reshape(1, LAT), W3n, b3n.reshape(1, DOUT))


def kernel(n_feats, e_feats, edge_index, W1e, b1e, W2e, b2e, W3e, b3e,
           W1n, b1n, W2n, b2n, W3n, b3n):
    N, DIN = n_feats.shape
    E, EIN = e_feats.shape
    LAT = W1e.shape[1]

    ei_r = edge_index.reshape(2, E // GRP, GRP)
    e_r = e_feats.reshape(E // 8, EIN * 8)
    W1e_e = W1e[:EIN]
    W1e_s = W1e[EIN:EIN + DIN]
    W1e_d = W1e[EIN + DIN:]
    W1n_a = W1n[:DIN]
    W1n_b = W1n[DIN:]
    eye8 = jnp.eye(8, dtype=jnp.float32)
    BD1 = jnp.kron(eye8, W1e_e)
    b1t = jnp.tile(b1e, 8)
    BD2 = jnp.kron(eye8, W2e)
    b2t = jnp.tile(b2e, 8)
    W3s = jax.scipy.linalg.block_diag(*([W3e] * 8))
    b3t = jnp.tile(b3e, 8)

    P, Q = _node_pre(n_feats, W1e_s, W1e_d, blk=1000)
    s_p, er = _sc_gather_add(ei_r, e_r, P, Q, E)
    e_out = _edge_mlp(er, s_p, BD1, b1t, BD2, b2t, W3s, b3t, LAT, blk=1000)
    sums_p, deg_p = _sc_segment_sum(ei_r, e_out, N, LAT)
    n_out = _node_mlp(sums_p, deg_p, n_feats, W1n_a, W1n_b, b1n,
                      W2n, b2n, W3n, b3n, blk=1000)
    return (n_out, e_out)


# speedup vs baseline: 2.3880x; 2.3880x over previous
import functools

import jax
import jax.numpy as jnp
from jax import lax
from jax.experimental import pallas as pl
from jax.experimental.pallas import tpu as pltpu
from jax.experimental.pallas import tpu_sc as plsc

NC = 2
NS = 16
NW = NC * NS
GRP = 128
GPC = 4
CHUNK = GRP * GPC


def _edge_mlp(e_r, s_p, BD1, b1t, BD2, b2t, W3s, b3t, LAT, blk):
    R, PK = e_r.shape
    E = R * 8
    DOUT = W3s.shape[1] // 8

    def body(e_ref, s_ref, w1, bb1, w2, bb2, w3, bb3, o_ref):
        x1 = jnp.maximum(
            jnp.dot(e_ref[...], w1[...], preferred_element_type=jnp.float32)
            + bb1[...] + s_ref[...],
            0.0,
        )
        x2 = jnp.maximum(
            jnp.dot(x1, w2[...], preferred_element_type=jnp.float32) + bb2[...],
            0.0,
        )
        z = jnp.dot(x2, w3[...], preferred_element_type=jnp.float32) + bb3[...]
        sg = jax.nn.sigmoid(z)
        for c in range(8):
            o_ref[:, c, :] = sg[:, c * DOUT:(c + 1) * DOUT]

    return pl.pallas_call(
        body,
        grid=(R // blk,),
        in_specs=[
            pl.BlockSpec((blk, PK), lambda i: (i, 0)),
            pl.BlockSpec((blk, PK), lambda i: (i, 0)),
            pl.BlockSpec((PK, PK), lambda i: (0, 0)),
            pl.BlockSpec((1, PK), lambda i: (0, 0)),
            pl.BlockSpec((PK, PK), lambda i: (0, 0)),
            pl.BlockSpec((1, PK), lambda i: (0, 0)),
            pl.BlockSpec((PK, 8 * DOUT), lambda i: (0, 0)),
            pl.BlockSpec((1, 8 * DOUT), lambda i: (0, 0)),
        ],
        out_specs=pl.BlockSpec((blk, 8, DOUT), lambda i: (i, 0, 0)),
        out_shape=jax.ShapeDtypeStruct((R, 8, DOUT), jnp.float32),
    )(e_r, s_p, BD1, b1t.reshape(1, PK), BD2, b2t.reshape(1, PK),
      W3s, b3t.reshape(1, 8 * DOUT)).reshape(E, DOUT)


def _node_pre(n_feats, W1e_s, W1e_d, blk):
    N, DIN = n_feats.shape
    LAT = W1e_s.shape[1]

    def body(n_ref, ws_ref, wd_ref, p_ref, q_ref):
        x = n_ref[...]
        p_ref[...] = jnp.dot(x, ws_ref[...], preferred_element_type=jnp.float32)
        q_ref[...] = jnp.dot(x, wd_ref[...], preferred_element_type=jnp.float32)

    return pl.pallas_call(
        body,
        grid=(N // blk,),
        in_specs=[
            pl.BlockSpec((blk, DIN), lambda i: (i, 0)),
            pl.BlockSpec((DIN, LAT), lambda i: (0, 0)),
            pl.BlockSpec((DIN, LAT), lambda i: (0, 0)),
        ],
        out_specs=[
            pl.BlockSpec((blk, LAT), lambda i: (i, 0)),
            pl.BlockSpec((blk, LAT), lambda i: (i, 0)),
        ],
        out_shape=[
            jax.ShapeDtypeStruct((N, LAT), jnp.float32),
            jax.ShapeDtypeStruct((N, LAT), jnp.float32),
        ],
    )(n_feats, W1e_s, W1e_d)


def _sc_gather_add(ei_r, P, Q, E):
    LAT = P.shape[1]
    PK = LAT * 8
    R = E // 8
    RPC = CHUNK // 8
    n_chunks = E // CHUNK
    iters = -(-n_chunks // NW)
    mesh = plsc.VectorSubcoreMesh(core_axis_name="c", subcore_axis_name="s")

    @functools.partial(
        pl.kernel,
        out_type=jax.ShapeDtypeStruct((R, PK), jnp.float32),
        mesh=mesh,
        scratch_types=[
            pltpu.VMEM((GPC, GRP), jnp.int32),
            pltpu.VMEM((GPC, GRP), jnp.int32),
            pltpu.VMEM((GPC, GRP), jnp.int32),
            pltpu.VMEM((GPC, GRP), jnp.int32),
            pltpu.VMEM((RPC, PK), jnp.float32),
            pltpu.VMEM((RPC, PK), jnp.float32),
            pltpu.VMEM((CHUNK, LAT), jnp.float32),
            pltpu.VMEM((CHUNK, LAT), jnp.float32),
            pltpu.VMEM((CHUNK, LAT), jnp.float32),
            pltpu.VMEM((CHUNK, LAT), jnp.float32),
            pltpu.SemaphoreType.DMA,
            pltpu.SemaphoreType.DMA,
            pltpu.SemaphoreType.DMA,
            pltpu.SemaphoreType.DMA,
        ],
        compiler_params=pltpu.CompilerParams(use_tc_tiling_on_sc=False),
    )
    def k(ei_hbm, p_hbm, q_hbm, s_hbm, isrc0, isrc1, idst0, idst1,
          sb0, sb1, pb0, pb1, qb0, qb1, semI, semG, semW0, semW1):
        wid = lax.axis_index("s") * NC + lax.axis_index("c")
        isrc = [isrc0, isrc1]
        idst = [idst0, idst1]
        sb = [sb0, sb1]
        pb = [pb0, pb1]
        qb = [qb0, qb1]
        semW = [semW0, semW1]

        def issue_idx(cid, b):
            g0 = cid * GPC
            pltpu.async_copy(ei_hbm.at[0, pl.ds(g0, GPC)], isrc[b], semI)
            pltpu.async_copy(ei_hbm.at[1, pl.ds(g0, GPC)], idst[b], semI)

        @pl.when(wid < n_chunks)
        def _():
            issue_idx(wid, 0)

        def outer(io, carry):
            for b in range(2):
                i = io * 2 + b
                cid = wid + i * NW

                @pl.when(cid < n_chunks)
                def _(i=i, b=b, cid=cid):
                    g0 = cid * GPC
                    pltpu.make_async_copy(
                        ei_hbm.at[0, pl.ds(g0, GPC)], isrc[b], semI).wait()
                    pltpu.make_async_copy(
                        ei_hbm.at[1, pl.ds(g0, GPC)], idst[b], semI).wait()
                    ncid = cid + NW

                    @pl.when(ncid < n_chunks)
                    def _():
                        issue_idx(ncid, 1 - b)

                    gs = []
                    for g in range(GPC):
                        sl = pl.ds(g * GRP, GRP)
                        gs.append(pltpu.async_copy(
                            p_hbm.at[isrc[b].at[g]], pb[b].at[sl], semG))
                        gs.append(pltpu.async_copy(
                            q_hbm.at[idst[b].at[g]], qb[b].at[sl], semG))
                    for h in gs:
                        h.wait()

                    @pl.when(i >= 2)
                    def _():
                        pltpu.make_async_copy(
                            s_hbm.at[pl.ds(0, RPC)], sb[b], semW[b]).wait()

                    def rows(j, c2):
                        for u in range(8):
                            r = j * 8 + u
                            sb[b][j, pl.ds(u * LAT, LAT)] = pb[b][r] + qb[b][r]
                        return c2

                    lax.fori_loop(0, RPC, rows, 0)
                    pltpu.async_copy(
                        sb[b], s_hbm.at[pl.ds(cid * RPC, RPC)], semW[b])

            return carry

        lax.fori_loop(0, (iters + 1) // 2, outer, 0)
        for b in range(2):
            pltpu.make_async_copy(s_hbm.at[pl.ds(0, RPC)], sb[b], semW[b]).wait()

    return k(ei_r, P, Q)


def _sc_segment_sum(ei_r, e_out, N, LAT):
    E, DOUT = e_out.shape
    HALF = DOUT // NC
    GPC2 = 4
    CH2 = GPC2 * GRP
    n_chunks = E // CH2
    iters = -(-n_chunks // NS)
    rs = (N // NS) // 8 * 8
    rs_last = N - (NS - 1) * rs
    ZR = 16
    nz = N // ZR
    mesh = plsc.VectorSubcoreMesh(core_axis_name="c", subcore_axis_name="s")

    @functools.partial(
        pl.kernel,
        out_type=[
            jax.ShapeDtypeStruct((N, DOUT), jnp.float32),
            jax.ShapeDtypeStruct((NC, N, LAT), jnp.float32),
        ],
        mesh=mesh,
        scratch_types=[
            pltpu.VMEM((GPC2, GRP), jnp.int32),
            pltpu.VMEM((GPC2, GRP), jnp.int32),
            pltpu.VMEM((CH2, HALF), jnp.float32),
            pltpu.VMEM((CH2, HALF), jnp.float32),
            pltpu.VMEM((GRP, LAT), jnp.float32),
            pltpu.VMEM((ZR, HALF), jnp.float32),
            pltpu.VMEM((ZR, LAT), jnp.float32),
            pltpu.VMEM_SHARED((N, HALF), jnp.float32),
            pltpu.VMEM_SHARED((N, LAT), jnp.float32),
            pltpu.SemaphoreType.DMA,
        ],
        compiler_params=pltpu.CompilerParams(use_tc_tiling_on_sc=False),
    )
    def k(ei_hbm, eout_hbm, sums_hbm, deg_hbm,
          idxb0, idxb1, rows0, rows1, onesb, zbs, zbd, ssh, dsh, sem):
        c = lax.axis_index("c")
        s = lax.axis_index("s")
        idxb = [idxb0, idxb1]
        rows = [rows0, rows1]

        zero16 = jnp.zeros((16,), jnp.float32)
        one16 = jnp.ones((16,), jnp.float32)
        for r in range(ZR):
            for c8 in range(HALF // 16):
                zbs[r, pl.ds(c8 * 16, 16)] = zero16
            for c8 in range(LAT // 16):
                zbd[r, pl.ds(c8 * 16, 16)] = zero16
        for r in range(GRP):
            for c8 in range(LAT // 16):
                onesb[r, pl.ds(c8 * 16, 16)] = one16

        def zbody(i, carry):
            j = s + i * NS

            @pl.when(j < nz)
            def _():
                pltpu.sync_copy(zbs, ssh.at[pl.ds(j * ZR, ZR)])
                pltpu.sync_copy(zbd, dsh.at[pl.ds(j * ZR, ZR)])

            return carry

        lax.fori_loop(0, -(-nz // NS), zbody, 0)
        plsc.subcore_barrier()

        def issue_reads(cid, b):
            g0 = cid * GPC2
            pltpu.async_copy(ei_hbm.at[1, pl.ds(g0, GPC2)], idxb[b], sem)
            pltpu.async_copy(
                eout_hbm.at[pl.ds(cid * CH2, CH2), pl.ds(c * HALF, HALF)],
                rows[b], sem)

        @pl.when(s < n_chunks)
        def _():
            issue_reads(s, 0)

        def outer(io, carry):
            for b in range(2):
                i = io * 2 + b
                cid = s + i * NS

                @pl.when(cid < n_chunks)
                def _(i=i, b=b, cid=cid):
                    g0 = cid * GPC2
                    pltpu.make_async_copy(
                        ei_hbm.at[1, pl.ds(g0, GPC2)], idxb[b], sem).wait()
                    pltpu.make_async_copy(
                        eout_hbm.at[pl.ds(cid * CH2, CH2),
                                    pl.ds(c * HALF, HALF)],
                        rows[b], sem).wait()
                    ncid = cid + NS

                    @pl.when(ncid < n_chunks)
                    def _():
                        issue_reads(ncid, 1 - b)

                    for g in range(GPC2):
                        sl = pl.ds(g * GRP, GRP)
                        pltpu.sync_copy(rows[b].at[sl], ssh.at[idxb[b].at[g]],
                                        add=True)

                    @pl.when(lax.rem(cid, NC) == c)
                    def _():
                        for g in range(GPC2):
                            pltpu.sync_copy(onesb, dsh.at[idxb[b].at[g]],
                                            add=True)

            return carry

        lax.fori_loop(0, (iters + 1) // 2, outer, 0)
        plsc.subcore_barrier()

        @pl.when(s < NS - 1)
        def _():
            pltpu.sync_copy(ssh.at[pl.ds(s * rs, rs)],
                            sums_hbm.at[pl.ds(s * rs, rs), pl.ds(c * HALF, HALF)])

        @pl.when(s == NS - 1)
        def _():
            base = (NS - 1) * rs
            pltpu.sync_copy(ssh.at[pl.ds(base, rs_last)],
                            sums_hbm.at[pl.ds(base, rs_last), pl.ds(c * HALF, HALF)])

        @pl.when(s < NS - 1)
        def _():
            pltpu.sync_copy(dsh.at[pl.ds(s * rs, rs)],
                            deg_hbm.at[c, pl.ds(s * rs, rs)])

        @pl.when(s == NS - 1)
        def _():
            base = (NS - 1) * rs
            pltpu.sync_copy(dsh.at[pl.ds(base, rs_last)],
                            deg_hbm.at[c, pl.ds(base, rs_last)])

    return k(ei_r, e_out)


def _node_mlp(sums_p, deg_p, n_feats, W1n_a, W1n_b, b1n, W2n, b2n, W3n, b3n, blk):
    N, DIN = n_feats.shape
    LAT = W1n_a.shape[1]
    DOUT = W3n.shape[1]

    def body(sp, dp, nf, w1a, w1b, b1, w2, b2, w3, b3, o_ref):
        sums = sp[...]
        deg = dp[0] + dp[1]
        hN = sums / jnp.maximum(deg[:, :1], 1.0)
        h = jnp.maximum(
            jnp.dot(nf[...], w1a[...], preferred_element_type=jnp.float32)
            + jnp.dot(hN, w1b[...], preferred_element_type=jnp.float32)
            + b1[...],
            0.0,
        )
        h = jnp.maximum(
            jnp.dot(h, w2[...], preferred_element_type=jnp.float32) + b2[...], 0.0
        )
        o_ref[...] = jax.nn.sigmoid(
            jnp.dot(h, w3[...], preferred_element_type=jnp.float32) + b3[...]
        )

    DSUM = sums_p.shape[1]
    DDEG = deg_p.shape[2]
    return pl.pallas_call(
        body,
        grid=(N // blk,),
        in_specs=[
            pl.BlockSpec((blk, DSUM), lambda i: (i, 0)),
            pl.BlockSpec((NC, blk, DDEG), lambda i: (0, i, 0)),
            pl.BlockSpec((blk, DIN), lambda i: (i, 0)),
            pl.BlockSpec((DIN, LAT), lambda i: (0, 0)),
            pl.BlockSpec((DSUM, LAT), lambda i: (0, 0)),
            pl.BlockSpec((1, LAT), lambda i: (0, 0)),
            pl.BlockSpec((LAT, LAT), lambda i: (0, 0)),
            pl.BlockSpec((1, LAT), lambda i: (0, 0)),
            pl.BlockSpec((LAT, DOUT), lambda i: (0, 0)),
            pl.BlockSpec((1, DOUT), lambda i: (0, 0)),
        ],
        out_specs=pl.BlockSpec((blk, DOUT), lambda i: (i, 0)),
        out_shape=jax.ShapeDtypeStruct((N, DOUT), jnp.float32),
    )(sums_p, deg_p, n_feats, W1n_a, W1n_b, b1n.reshape(1, LAT), W2n,
      b2n.reshape(1, LAT), W3n, b3n.reshape(1, DOUT))


def kernel(n_feats, e_feats, edge_index, W1e, b1e, W2e, b2e, W3e, b3e,
           W1n, b1n, W2n, b2n, W3n, b3n):
    N, DIN = n_feats.shape
    E, EIN = e_feats.shape
    LAT = W1e.shape[1]

    ei_r = edge_index.reshape(2, E // GRP, GRP)
    e_r = e_feats.reshape(E // 8, EIN * 8)
    W1e_e = W1e[:EIN]
    W1e_s = W1e[EIN:EIN + DIN]
    W1e_d = W1e[EIN + DIN:]
    W1n_a = W1n[:DIN]
    W1n_b = W1n[DIN:]
    eye8 = jnp.eye(8, dtype=jnp.float32)
    BD1 = jnp.kron(eye8, W1e_e)
    b1t = jnp.tile(b1e, 8)
    BD2 = jnp.kron(eye8, W2e)
    b2t = jnp.tile(b2e, 8)
    W3s = jax.scipy.linalg.block_diag(*([W3e] * 8))
    b3t = jnp.tile(b3e, 8)

    P, Q = _node_pre(n_feats, W1e_s, W1e_d, blk=1000)
    s_p = _sc_gather_add(ei_r, P, Q, E)
    e_out = _edge_mlp(e_r, s_p, BD1, b1t, BD2, b2t, W3s, b3t, LAT, blk=2000)
    sums_p, deg_p = _sc_segment_sum(ei_r, e_out, N, LAT)
    n_out = _node_mlp(sums_p, deg_p, n_feats, W1n_a, W1n_b, b1n,
                      W2n, b2n, W3n, b3n, blk=1000)
    return (n_out, e_out)
